# baseline (speedup 1.0000x reference)
import functools
import jax
import jax.numpy as jnp
from jax import lax
from jax.experimental import pallas as pl
from jax.experimental.pallas import tpu as pltpu
from jax.experimental.pallas import tpu_sc as plsc

NC, NS = 2, 16
NW = NC * NS
B = 16384
R = B // NW
MD = 64
GD = 16
LN = 128


def _prep(idx2):
    order = jnp.argsort(idx2, axis=1).astype(jnp.int32)
    sv = jnp.take_along_axis(idx2, order, axis=1)
    col = (sv >> 7).astype(jnp.int32)
    lane = (sv & 127).astype(jnp.int32)
    k = jnp.arange(B, dtype=jnp.int32)[None, :]
    newblk = ((k % R) == 0) | (col != jnp.roll(col, 1, axis=1))
    slot = jnp.cumsum(newblk.reshape(2, NW, R).astype(jnp.int32), axis=2) - 1
    blk = jnp.zeros((2, NW, R), jnp.int32)
    brow = jnp.repeat(jnp.arange(2, dtype=jnp.int32), NW * R)
    wrow = jnp.tile(jnp.repeat(jnp.arange(NW, dtype=jnp.int32), R), 2)
    blk = blk.at[brow, wrow, slot.reshape(-1)].set(col.reshape(-1))
    aux = (slot << 7) | lane.reshape(2, NW, R)
    pos = order.reshape(2, NW, 4, LN)
    return aux, blk, pos


NBUF = 4


def _phase(aux_s, blk_s, ttm, ttg, out_hbm, posv, rows, mring, gring,
           semm, semg, ssc, wid):
    def fire(s):
        c = blk_s[pl.ds(jnp.minimum(s, R - 1), 16)][0]
        off = pl.multiple_of(c * LN, LN)
        par = s & (NBUF - 1)
        for p in range(NBUF):
            @pl.when(par == p)
            def _():
                pltpu.async_copy(ttm.at[:, pl.ds(off, LN)], mring.at[p],
                                 semm[p])
                pltpu.async_copy(ttg.at[:, pl.ds(off, LN)], gring.at[p],
                                 semg[p])

    def wait(s):
        par = s & (NBUF - 1)
        for p in range(NBUF):
            @pl.when(par == p)
            def _():
                pltpu.make_async_copy(ttm.at[:, pl.ds(0, LN)], mring.at[p],
                                      semm[p]).wait()
                pltpu.make_async_copy(ttg.at[:, pl.ds(0, LN)], gring.at[p],
                                      semg[p]).wait()

    for s0 in range(NBUF - 1):
        fire(s0)

    jv = [lax.iota(jnp.int32, 16) + 16 * q for q in range(4)]
    jg = lax.iota(jnp.int32, 16)

    def row_body(r, cur):
        a = aux_s[pl.ds(r, 16)][0]
        s = a >> 7
        l = a & 127

        @pl.when(s > cur)
        def _():
            wait(s)
            fire(s + NBUF - 1)

        pv = jnp.full((16,), s & (NBUF - 1), jnp.int32)
        lv = jnp.full((16,), l, jnp.int32)
        for q in range(4):
            v = plsc.load_gather(mring, [pv, jv[q], lv])
            rows[r, pl.ds(16 * q, 16)] = v
        rows[r, pl.ds(MD, 16)] = plsc.load_gather(gring, [pv, jg, lv])
        return s

    last = lax.fori_loop(0, R, row_body, jnp.int32(-1))
    for d in range(1, NBUF):
        wait(last + d)

    for q in range(4):
        pltpu.async_copy(rows.at[pl.ds(q * LN, LN)],
                         out_hbm.at[posv.at[q]], ssc).wait()


def _sc_body(aux_u, blk_u, pos_u, aux_i, blk_i, pos_i,
             ttum, ttim, ttug, ttig,
             out_u, out_i,
             aux_s, blk_s, posv, rows, mring, gring,
             sm0, sm1, sm2, sm3, sg0, sg1, sg2, sg3, ssc):
    wid = lax.axis_index("s") * NC + lax.axis_index("c")
    semm = (sm0, sm1, sm2, sm3)
    semg = (sg0, sg1, sg2, sg3)

    pltpu.sync_copy(aux_u.at[wid], aux_s.at[pl.ds(0, R)])
    pltpu.sync_copy(blk_u.at[wid], blk_s.at[pl.ds(0, R)])
    pltpu.sync_copy(pos_u.at[wid], posv)
    _phase(aux_s, blk_s, ttum, ttug, out_u, posv, rows, mring, gring,
           semm, semg, ssc, wid)

    pltpu.sync_copy(aux_i.at[wid], aux_s.at[pl.ds(0, R)])
    pltpu.sync_copy(blk_i.at[wid], blk_s.at[pl.ds(0, R)])
    pltpu.sync_copy(pos_i.at[wid], posv)
    _phase(aux_s, blk_s, ttim, ttig, out_i, posv, rows, mring, gring,
           semm, semg, ssc, wid)


@functools.cache
def _sc_gather():
    mesh = plsc.VectorSubcoreMesh(core_axis_name="c", subcore_axis_name="s",
                                  num_cores=NC, num_subcores=NS)
    return pl.kernel(
        _sc_body,
        out_type=(
            jax.ShapeDtypeStruct((B, LN), jnp.float32),
            jax.ShapeDtypeStruct((B, LN), jnp.float32),
        ),
        mesh=mesh,
        compiler_params=pltpu.CompilerParams(use_tc_tiling_on_sc=True,
                                             needs_layout_passes=False),
        scratch_types=(
            pltpu.VMEM((R + 16,), jnp.int32),
            pltpu.VMEM((R + 16,), jnp.int32),
            pltpu.VMEM((4, LN), jnp.int32),
            pltpu.VMEM((R, LN), jnp.float32),
            pltpu.VMEM((NBUF, MD, LN), jnp.float32),
            pltpu.VMEM((NBUF, GD, LN), jnp.float32),
        ) + (pltpu.SemaphoreType.DMA,) * 9,
    )


BM = 2048


def _mlp_body(ou_ref, oi_ref, w1a_ref, w1b_ref, b1_ref, w2_ref, b2_ref,
              w3_ref, b3_ref, wfh_ref, wfg_ref, bf_ref, out_ref):
    ou = ou_ref[...]
    oi = oi_ref[...]
    xu = ou[:, 0:MD]
    xi = oi[:, 0:MD]
    g = ou[:, MD:MD + GD] * oi[:, MD:MD + GD]
    h = xu @ w1a_ref[...] + xi @ w1b_ref[...] + b1_ref[...]
    h = jnp.maximum(h, 0.0)
    h = jnp.maximum(h @ w2_ref[...] + b2_ref[...], 0.0)
    h = jnp.maximum(h @ w3_ref[...] + b3_ref[...], 0.0)
    s = h @ wfh_ref[...] + g @ wfg_ref[...] + bf_ref[0, 0]
    out_ref[...] = s


def _rep(shape):
    nd = len(shape)
    return pl.BlockSpec(shape, lambda i: (0,) * nd)


_mlp = pl.pallas_call(
    _mlp_body,
    grid=(B // BM,),
    in_specs=[
        pl.BlockSpec((BM, LN), lambda i: (i, 0)),
        pl.BlockSpec((BM, LN), lambda i: (i, 0)),
        _rep((MD, 64)), _rep((64, 64)), _rep((1, 64)),
        _rep((64, 32)), _rep((1, 32)),
        _rep((32, 16)), _rep((1, 16)),
        _rep((16, 1)), _rep((16, 1)), _rep((1, 1)),
    ],
    out_specs=pl.BlockSpec((BM, 1), lambda i: (i, 0)),
    out_shape=jax.ShapeDtypeStruct((B, 1), jnp.float32),
)


def kernel(user, item, embed_user_GMF, embed_item_GMF, embed_user_MLP,
           embed_item_MLP, W1, b1, W2, b2, W3, b3, Wf, bf):
    user = user.astype(jnp.int32)
    item = item.astype(jnp.int32)
    aux, blk, pos = _prep(jnp.stack([user, item]))
    out_u, out_i = _sc_gather()(
        aux[0], blk[0], pos[0], aux[1], blk[1], pos[1],
        embed_user_MLP.T, embed_item_MLP.T,
        embed_user_GMF.T, embed_item_GMF.T)
    score = _mlp(out_u, out_i,
                 W1[:MD], W1[MD:], b1.reshape(1, -1),
                 W2, b2.reshape(1, -1),
                 W3, b3.reshape(1, -1),
                 Wf[:GD], Wf[GD:], bf.reshape(1, 1))
    return score.reshape(-1)


# speedup vs baseline: 1.2324x; 1.2324x over previous
import functools
import jax
import jax.numpy as jnp
from jax import lax
from jax.experimental import pallas as pl
from jax.experimental.pallas import tpu as pltpu
from jax.experimental.pallas import tpu_sc as plsc

NC, NS = 2, 16
NW = NC * NS
B = 16384
R = B // NW
MD = 64
GD = 16
LN = 128


def _prep(idx):
    order = jnp.argsort(idx).astype(jnp.int32)
    sv = jnp.take(idx, order)
    col = (sv >> 7).astype(jnp.int32)
    lane = (sv & 127).astype(jnp.int32)
    k = jnp.arange(B, dtype=jnp.int32)
    newblk = ((k % R) == 0) | (col != jnp.roll(col, 1))
    slot = jnp.cumsum(newblk.reshape(NW, R).astype(jnp.int32), axis=1) - 1
    blk = jnp.zeros((NW, R), jnp.int32)
    wrow = jnp.repeat(jnp.arange(NW, dtype=jnp.int32), R)
    blk = blk.at[wrow, slot.reshape(-1)].set(col)
    aux = (slot << 7) | lane.reshape(NW, R)
    pos = order.reshape(NW, 4, LN)
    return aux, blk, pos


NBUF = 4


def _phase(aux_s, blk_s, ttm, ttg, out_hbm, posv, rows, mring, gring,
           semm, semg, ssc, wid):
    def fire(s):
        c = blk_s[pl.ds(jnp.minimum(s, R - 1), 16)][0]
        off = pl.multiple_of(c * LN, LN)
        par = s & (NBUF - 1)
        for p in range(NBUF):
            @pl.when(par == p)
            def _():
                pltpu.async_copy(ttm.at[:, pl.ds(off, LN)], mring.at[p],
                                 semm[p])
                pltpu.async_copy(ttg.at[:, pl.ds(off, LN)], gring.at[p],
                                 semg[p])

    def wait(s):
        par = s & (NBUF - 1)
        for p in range(NBUF):
            @pl.when(par == p)
            def _():
                pltpu.make_async_copy(ttm.at[:, pl.ds(0, LN)], mring.at[p],
                                      semm[p]).wait()
                pltpu.make_async_copy(ttg.at[:, pl.ds(0, LN)], gring.at[p],
                                      semg[p]).wait()

    for s0 in range(NBUF - 1):
        fire(s0)

    jv = [lax.iota(jnp.int32, 16) + 16 * q for q in range(4)]
    jg = lax.iota(jnp.int32, 16)

    def row_body(r, cur):
        a = aux_s[pl.ds(r, 16)][0]
        s = a >> 7
        l = a & 127

        @pl.when(s > cur)
        def _():
            wait(s)
            fire(s + NBUF - 1)

        pv = jnp.full((16,), s & (NBUF - 1), jnp.int32)
        lv = jnp.full((16,), l, jnp.int32)
        for q in range(4):
            v = plsc.load_gather(mring, [pv, jv[q], lv])
            rows[r, pl.ds(16 * q, 16)] = v
        rows[r, pl.ds(MD, 16)] = plsc.load_gather(gring, [pv, jg, lv])
        return s

    last = lax.fori_loop(0, R, row_body, jnp.int32(-1))
    for d in range(1, NBUF):
        wait(last + d)

    for q in range(4):
        pltpu.async_copy(rows.at[pl.ds(q * LN, LN)],
                         out_hbm.at[posv.at[q]], ssc).wait()


def _sc_body(aux_u, blk_u, pos_u, ttum, ttug,
             out_u,
             aux_s, blk_s, posv, rows, mring, gring,
             sm0, sm1, sm2, sm3, sg0, sg1, sg2, sg3, ssc):
    wid = lax.axis_index("s") * NC + lax.axis_index("c")
    semm = (sm0, sm1, sm2, sm3)
    semg = (sg0, sg1, sg2, sg3)

    pltpu.sync_copy(aux_u.at[wid], aux_s.at[pl.ds(0, R)])
    pltpu.sync_copy(blk_u.at[wid], blk_s.at[pl.ds(0, R)])
    pltpu.sync_copy(pos_u.at[wid], posv)
    _phase(aux_s, blk_s, ttum, ttug, out_u, posv, rows, mring, gring,
           semm, semg, ssc, wid)


@functools.cache
def _sc_gather():
    mesh = plsc.VectorSubcoreMesh(core_axis_name="c", subcore_axis_name="s",
                                  num_cores=NC, num_subcores=NS)
    return pl.kernel(
        _sc_body,
        out_type=jax.ShapeDtypeStruct((B, LN), jnp.float32),
        mesh=mesh,
        compiler_params=pltpu.CompilerParams(use_tc_tiling_on_sc=True,
                                             needs_layout_passes=False),
        scratch_types=(
            pltpu.VMEM((R + 16,), jnp.int32),
            pltpu.VMEM((R + 16,), jnp.int32),
            pltpu.VMEM((4, LN), jnp.int32),
            pltpu.VMEM((R, LN), jnp.float32),
            pltpu.VMEM((NBUF, MD, LN), jnp.float32),
            pltpu.VMEM((NBUF, GD, LN), jnp.float32),
        ) + (pltpu.SemaphoreType.DMA,) * 9,
    )


BM = 2048


def _mlp_body(ou_ref, oi_ref, w1a_ref, w1b_ref, b1_ref, w2_ref, b2_ref,
              w3_ref, b3_ref, wfh_ref, wfg_ref, bf_ref, out_ref):
    ou = ou_ref[...]
    oi = oi_ref[...]
    xu = ou[:, 0:MD]
    xi = oi[:, 0:MD]
    g = ou[:, MD:MD + GD] * oi[:, MD:MD + GD]
    h = xu @ w1a_ref[...] + xi @ w1b_ref[...] + b1_ref[...]
    h = jnp.maximum(h, 0.0)
    h = jnp.maximum(h @ w2_ref[...] + b2_ref[...], 0.0)
    h = jnp.maximum(h @ w3_ref[...] + b3_ref[...], 0.0)
    s = h @ wfh_ref[...] + g @ wfg_ref[...] + bf_ref[0, 0]
    out_ref[...] = s


def _rep(shape):
    nd = len(shape)
    return pl.BlockSpec(shape, lambda i: (0,) * nd)


_mlp = pl.pallas_call(
    _mlp_body,
    grid=(B // BM,),
    in_specs=[
        pl.BlockSpec((BM, LN), lambda i: (i, 0)),
        pl.BlockSpec((BM, LN), lambda i: (i, 0)),
        _rep((MD, 64)), _rep((64, 64)), _rep((1, 64)),
        _rep((64, 32)), _rep((1, 32)),
        _rep((32, 16)), _rep((1, 16)),
        _rep((16, 1)), _rep((16, 1)), _rep((1, 1)),
    ],
    out_specs=pl.BlockSpec((BM, 1), lambda i: (i, 0)),
    out_shape=jax.ShapeDtypeStruct((B, 1), jnp.float32),
)


def kernel(user, item, embed_user_GMF, embed_item_GMF, embed_user_MLP,
           embed_item_MLP, W1, b1, W2, b2, W3, b3, Wf, bf):
    user = user.astype(jnp.int32)
    item = item.astype(jnp.int32)
    aux_u, blk_u, pos_u = _prep(user)
    out_u = _sc_gather()(aux_u, blk_u, pos_u,
                         embed_user_MLP.T, embed_user_GMF.T)
    aux_i, blk_i, pos_i = _prep(item)
    out_i = _sc_gather()(aux_i, blk_i, pos_i,
                         embed_item_MLP.T, embed_item_GMF.T)
    score = _mlp(out_u, out_i,
                 W1[:MD], W1[MD:], b1.reshape(1, -1),
                 W2, b2.reshape(1, -1),
                 W3, b3.reshape(1, -1),
                 Wf[:GD], Wf[GD:], bf.reshape(1, 1))
    return score.reshape(-1)


# speedup vs baseline: 1.2583x; 1.0211x over previous
import functools
import jax
import jax.numpy as jnp
from jax import lax
from jax.experimental import pallas as pl
from jax.experimental.pallas import tpu as pltpu
from jax.experimental.pallas import tpu_sc as plsc

NC, NS = 2, 16
NW = NC * NS
B = 16384
R = B // NW
MD = 64
GD = 16
LN = 128


def _prep(idx):
    order = jnp.argsort(idx).astype(jnp.int32)
    sv = jnp.take(idx, order)
    col = (sv >> 7).astype(jnp.int32)
    lane = (sv & 127).astype(jnp.int32)
    k = jnp.arange(B, dtype=jnp.int32)
    newblk = ((k % R) == 0) | (col != jnp.roll(col, 1))
    slot = jnp.cumsum(newblk.reshape(NW, R).astype(jnp.int32), axis=1) - 1
    blk = jnp.zeros((NW, R), jnp.int32)
    wrow = jnp.repeat(jnp.arange(NW, dtype=jnp.int32), R)
    blk = blk.at[wrow, slot.reshape(-1)].set(col)
    aux = (slot << 7) | lane.reshape(NW, R)
    pos = order.reshape(NW, 4, LN)
    return aux, blk, pos


NBUF = 5


def _phase(aux_s, blk_s, ttm, ttg, out_hbm, posv, rows, mring, gring,
           semm, semg, ssc, wid):
    def fire(s):
        c = blk_s[pl.ds(jnp.minimum(s, R - 1), 16)][0]
        off = pl.multiple_of(c * LN, LN)
        par = s % NBUF
        for p in range(NBUF):
            @pl.when(par == p)
            def _():
                pltpu.async_copy(ttm.at[:, pl.ds(off, LN)], mring.at[p],
                                 semm[p])
                pltpu.async_copy(ttg.at[:, pl.ds(off, LN)], gring.at[p],
                                 semg[p])

    def wait(s):
        par = s % NBUF
        for p in range(NBUF):
            @pl.when(par == p)
            def _():
                pltpu.make_async_copy(ttm.at[:, pl.ds(0, LN)], mring.at[p],
                                      semm[p]).wait()
                pltpu.make_async_copy(ttg.at[:, pl.ds(0, LN)], gring.at[p],
                                      semg[p]).wait()

    for s0 in range(NBUF - 1):
        fire(s0)

    jv = [lax.iota(jnp.int32, 16) + 16 * q for q in range(4)]
    jg = lax.iota(jnp.int32, 16)

    def row_body(r, cur):
        a = aux_s[pl.ds(r, 16)][0]
        s = a >> 7
        l = a & 127

        @pl.when(s > cur)
        def _():
            wait(s)
            fire(s + NBUF - 1)

        pv = jnp.full((16,), s % NBUF, jnp.int32)
        lv = jnp.full((16,), l, jnp.int32)
        for q in range(4):
            v = plsc.load_gather(mring, [pv, jv[q], lv])
            rows[r, pl.ds(16 * q, 16)] = v
        rows[r, pl.ds(MD, 16)] = plsc.load_gather(gring, [pv, jg, lv])
        return s

    last = lax.fori_loop(0, R, row_body, jnp.int32(-1))
    for d in range(1, NBUF):
        wait(last + d)

    for q in range(4):
        pltpu.async_copy(rows.at[pl.ds(q * LN, LN)],
                         out_hbm.at[posv.at[q]], ssc).wait()


def _sc_body(aux_u, blk_u, pos_u, ttum, ttug,
             out_u,
             aux_s, blk_s, posv, rows, mring, gring,
             sm0, sm1, sm2, sm3, sm4, sg0, sg1, sg2, sg3, sg4, ssc):
    wid = lax.axis_index("s") * NC + lax.axis_index("c")
    semm = (sm0, sm1, sm2, sm3, sm4)
    semg = (sg0, sg1, sg2, sg3, sg4)

    pltpu.sync_copy(aux_u.at[wid], aux_s.at[pl.ds(0, R)])
    pltpu.sync_copy(blk_u.at[wid], blk_s.at[pl.ds(0, R)])
    pltpu.sync_copy(pos_u.at[wid], posv)
    _phase(aux_s, blk_s, ttum, ttug, out_u, posv, rows, mring, gring,
           semm, semg, ssc, wid)


@functools.cache
def _sc_gather():
    mesh = plsc.VectorSubcoreMesh(core_axis_name="c", subcore_axis_name="s",
                                  num_cores=NC, num_subcores=NS)
    return pl.kernel(
        _sc_body,
        out_type=jax.ShapeDtypeStruct((B, LN), jnp.float32),
        mesh=mesh,
        compiler_params=pltpu.CompilerParams(use_tc_tiling_on_sc=True,
                                             needs_layout_passes=False),
        scratch_types=(
            pltpu.VMEM((R + 16,), jnp.int32),
            pltpu.VMEM((R + 16,), jnp.int32),
            pltpu.VMEM((4, LN), jnp.int32),
            pltpu.VMEM((R, LN), jnp.float32),
            pltpu.VMEM((NBUF, MD, LN), jnp.float32),
            pltpu.VMEM((NBUF, GD, LN), jnp.float32),
        ) + (pltpu.SemaphoreType.DMA,) * 11,
    )


BM = 2048


def _mlp_body(ou_ref, oi_ref, w1a_ref, w1b_ref, b1_ref, w2_ref, b2_ref,
              w3_ref, b3_ref, wfh_ref, wfg_ref, bf_ref, out_ref):
    ou = ou_ref[...]
    oi = oi_ref[...]
    xu = ou[:, 0:MD]
    xi = oi[:, 0:MD]
    g = ou[:, MD:MD + GD] * oi[:, MD:MD + GD]
    h = xu @ w1a_ref[...] + xi @ w1b_ref[...] + b1_ref[...]
    h = jnp.maximum(h, 0.0)
    h = jnp.maximum(h @ w2_ref[...] + b2_ref[...], 0.0)
    h = jnp.maximum(h @ w3_ref[...] + b3_ref[...], 0.0)
    s = h @ wfh_ref[...] + g @ wfg_ref[...] + bf_ref[0, 0]
    out_ref[...] = s


def _rep(shape):
    nd = len(shape)
    return pl.BlockSpec(shape, lambda i: (0,) * nd)


_mlp = pl.pallas_call(
    _mlp_body,
    grid=(B // BM,),
    in_specs=[
        pl.BlockSpec((BM, LN), lambda i: (i, 0)),
        pl.BlockSpec((BM, LN), lambda i: (i, 0)),
        _rep((MD, 64)), _rep((64, 64)), _rep((1, 64)),
        _rep((64, 32)), _rep((1, 32)),
        _rep((32, 16)), _rep((1, 16)),
        _rep((16, 1)), _rep((16, 1)), _rep((1, 1)),
    ],
    out_specs=pl.BlockSpec((BM, 1), lambda i: (i, 0)),
    out_shape=jax.ShapeDtypeStruct((B, 1), jnp.float32),
)


def kernel(user, item, embed_user_GMF, embed_item_GMF, embed_user_MLP,
           embed_item_MLP, W1, b1, W2, b2, W3, b3, Wf, bf):
    user = user.astype(jnp.int32)
    item = item.astype(jnp.int32)
    aux_u, blk_u, pos_u = _prep(user)
    out_u = _sc_gather()(aux_u, blk_u, pos_u,
                         embed_user_MLP.T, embed_user_GMF.T)
    aux_i, blk_i, pos_i = _prep(item)
    out_i = _sc_gather()(aux_i, blk_i, pos_i,
                         embed_item_MLP.T, embed_item_GMF.T)
    score = _mlp(out_u, out_i,
                 W1[:MD], W1[MD:], b1.reshape(1, -1),
                 W2, b2.reshape(1, -1),
                 W3, b3.reshape(1, -1),
                 Wf[:GD], Wf[GD:], bf.reshape(1, 1))
    return score.reshape(-1)


# speedup vs baseline: 1.2607x; 1.0019x over previous
import functools
import jax
import jax.numpy as jnp
from jax import lax
from jax.experimental import pallas as pl
from jax.experimental.pallas import tpu as pltpu
from jax.experimental.pallas import tpu_sc as plsc

NC, NS = 2, 16
NW = NC * NS
B = 16384
R = B // NW
MD = 64
GD = 16
LN = 128


def _prep(idx):
    order = jnp.argsort(idx).astype(jnp.int32)
    sv = jnp.take(idx, order)
    col = (sv >> 7).astype(jnp.int32)
    lane = (sv & 127).astype(jnp.int32)
    k = jnp.arange(B, dtype=jnp.int32)
    newblk = ((k % R) == 0) | (col != jnp.roll(col, 1))
    slot = jnp.cumsum(newblk.reshape(NW, R).astype(jnp.int32), axis=1) - 1
    blk = jnp.zeros((NW, R), jnp.int32)
    wrow = jnp.repeat(jnp.arange(NW, dtype=jnp.int32), R)
    blk = blk.at[wrow, slot.reshape(-1)].set(col)
    aux = (slot << 7) | lane.reshape(NW, R)
    pos = order.reshape(NW, 4, LN)
    return aux, blk, pos


NBUF = 5


def _phase(aux_s, blk_s, ttm, ttg, out_hbm, posv, rows, mring, gring,
           semm, semg, ssc, wid):
    def fire(s):
        c = blk_s[pl.ds(jnp.minimum(s, R - 1), 16)][0]
        off = pl.multiple_of(c * LN, LN)
        par = s % NBUF
        for p in range(NBUF):
            @pl.when(par == p)
            def _():
                pltpu.async_copy(ttm.at[:, pl.ds(off, LN)], mring.at[p],
                                 semm[p])
                pltpu.async_copy(ttg.at[:, pl.ds(off, LN)], gring.at[p],
                                 semg[p])

    def wait(s):
        par = s % NBUF
        for p in range(NBUF):
            @pl.when(par == p)
            def _():
                pltpu.make_async_copy(ttm.at[:, pl.ds(0, LN)], mring.at[p],
                                      semm[p]).wait()
                pltpu.make_async_copy(ttg.at[:, pl.ds(0, LN)], gring.at[p],
                                      semg[p]).wait()

    for s0 in range(NBUF - 1):
        fire(s0)

    jv = [lax.iota(jnp.int32, 16) + 16 * q for q in range(4)]
    jg = lax.iota(jnp.int32, 16)

    def row_body(r, cur):
        a = aux_s[pl.ds(r, 16)][0]
        s = a >> 7
        l = a & 127

        @pl.when(s > cur)
        def _():
            wait(s)
            fire(s + NBUF - 1)

        pv = jnp.full((16,), s % NBUF, jnp.int32)
        lv = jnp.full((16,), l, jnp.int32)
        for q in range(4):
            v = plsc.load_gather(mring, [pv, jv[q], lv])
            rows[r, pl.ds(16 * q, 16)] = v
        rows[r, pl.ds(MD, 16)] = plsc.load_gather(gring, [pv, jg, lv])
        return s

    last = lax.fori_loop(0, R, row_body, jnp.int32(-1))
    for d in range(1, NBUF):
        wait(last + d)

    for q in range(4):
        pltpu.async_copy(rows.at[pl.ds(q * LN, LN)],
                         out_hbm.at[posv.at[q]], ssc).wait()


def _sc_body(aux_u, blk_u, pos_u, ttum, ttug,
             out_u,
             aux_s, blk_s, posv, rows, mring, gring,
             sm0, sm1, sm2, sm3, sm4, sg0, sg1, sg2, sg3, sg4, ssc):
    wid = lax.axis_index("s") * NC + lax.axis_index("c")
    semm = (sm0, sm1, sm2, sm3, sm4)
    semg = (sg0, sg1, sg2, sg3, sg4)

    pltpu.sync_copy(aux_u.at[wid], aux_s.at[pl.ds(0, R)])
    pltpu.sync_copy(blk_u.at[wid], blk_s.at[pl.ds(0, R)])
    pltpu.sync_copy(pos_u.at[wid], posv)
    _phase(aux_s, blk_s, ttum, ttug, out_u, posv, rows, mring, gring,
           semm, semg, ssc, wid)


@functools.cache
def _sc_gather():
    mesh = plsc.VectorSubcoreMesh(core_axis_name="c", subcore_axis_name="s",
                                  num_cores=NC, num_subcores=NS)
    return pl.kernel(
        _sc_body,
        out_type=jax.ShapeDtypeStruct((B, LN), jnp.float32),
        mesh=mesh,
        compiler_params=pltpu.CompilerParams(use_tc_tiling_on_sc=True,
                                             needs_layout_passes=False),
        scratch_types=(
            pltpu.VMEM((R + 16,), jnp.int32),
            pltpu.VMEM((R + 16,), jnp.int32),
            pltpu.VMEM((4, LN), jnp.int32),
            pltpu.VMEM((R, LN), jnp.float32),
            pltpu.VMEM((NBUF, MD, LN), jnp.float32),
            pltpu.VMEM((NBUF, GD, LN), jnp.float32),
        ) + (pltpu.SemaphoreType.DMA,) * 11,
    )


BM = 2048


def _mlp_u_body(ou_ref, w1a_ref, b1_ref, pu_ref):
    ou = ou_ref[...]
    pu = ou[:, 0:MD] @ w1a_ref[...] + b1_ref[...]
    pu_ref[...] = jnp.concatenate([pu, ou[:, MD:MD + GD]], axis=1)


def _mlp_f_body(pu_ref, oi_ref, w1b_ref, w2_ref, b2_ref,
                w3_ref, b3_ref, wfh_ref, wfg_ref, bf_ref, out_ref):
    pu = pu_ref[...]
    oi = oi_ref[...]
    g = pu[:, MD:MD + GD] * oi[:, MD:MD + GD]
    h = jnp.maximum(pu[:, 0:MD] + oi[:, 0:MD] @ w1b_ref[...], 0.0)
    h = jnp.maximum(h @ w2_ref[...] + b2_ref[...], 0.0)
    h = jnp.maximum(h @ w3_ref[...] + b3_ref[...], 0.0)
    s = h @ wfh_ref[...] + g @ wfg_ref[...] + bf_ref[0, 0]
    out_ref[...] = s


def _rep(shape):
    nd = len(shape)
    return pl.BlockSpec(shape, lambda i: (0,) * nd)


_mlp_u = pl.pallas_call(
    _mlp_u_body,
    grid=(B // BM,),
    in_specs=[
        pl.BlockSpec((BM, LN), lambda i: (i, 0)),
        _rep((MD, 64)), _rep((1, 64)),
    ],
    out_specs=pl.BlockSpec((BM, MD + GD), lambda i: (i, 0)),
    out_shape=jax.ShapeDtypeStruct((B, MD + GD), jnp.float32),
)

_mlp_f = pl.pallas_call(
    _mlp_f_body,
    grid=(B // BM,),
    in_specs=[
        pl.BlockSpec((BM, MD + GD), lambda i: (i, 0)),
        pl.BlockSpec((BM, LN), lambda i: (i, 0)),
        _rep((64, 64)),
        _rep((64, 32)), _rep((1, 32)),
        _rep((32, 16)), _rep((1, 16)),
        _rep((16, 1)), _rep((16, 1)), _rep((1, 1)),
    ],
    out_specs=pl.BlockSpec((BM, 1), lambda i: (i, 0)),
    out_shape=jax.ShapeDtypeStruct((B, 1), jnp.float32),
)


def kernel(user, item, embed_user_GMF, embed_item_GMF, embed_user_MLP,
           embed_item_MLP, W1, b1, W2, b2, W3, b3, Wf, bf):
    user = user.astype(jnp.int32)
    item = item.astype(jnp.int32)
    aux_u, blk_u, pos_u = _prep(user)
    out_u = _sc_gather()(aux_u, blk_u, pos_u,
                         embed_user_MLP.T, embed_user_GMF.T)
    aux_i, blk_i, pos_i = _prep(item)
    out_i = _sc_gather()(aux_i, blk_i, pos_i,
                         embed_item_MLP.T, embed_item_GMF.T)
    pu = _mlp_u(out_u, W1[:MD], b1.reshape(1, -1))
    score = _mlp_f(pu, out_i,
                   W1[MD:],
                   W2, b2.reshape(1, -1),
                   W3, b3.reshape(1, -1),
                   Wf[:GD], Wf[GD:], bf.reshape(1, 1))
    return score.reshape(-1)
